# baseline (speedup 1.0000x reference)
import functools

import jax
import jax.numpy as jnp
from jax.experimental import pallas as pl

ROWS, COLS = 128, 100000
BLOCK_ROWS = 8


def _gumbel_softmax_block(logits_ref, u_ref, out_ref):
    g = logits_ref[...] - jnp.log(-jnp.log(u_ref[...]))
    m = jnp.max(g, axis=-1, keepdims=True)
    e = jnp.exp(g - m)
    s = jnp.sum(e, axis=-1, keepdims=True)
    out_ref[...] = e / s


@jax.jit
def kernel(logits, u):
    grid = (ROWS // BLOCK_ROWS,)
    spec = pl.BlockSpec((BLOCK_ROWS, COLS), lambda i: (i, 0))
    return pl.pallas_call(
        _gumbel_softmax_block,
        grid=grid,
        in_specs=[spec, spec],
        out_specs=spec,
        out_shape=jax.ShapeDtypeStruct((ROWS, COLS), jnp.float32),
    )(logits, u)
